# Initial kernel scaffold; baseline (speedup 1.0000x reference)
import jax
import jax.numpy as jnp
from jax.experimental import pallas as pl


def kernel(x, edge_index, batch, W1l, b1l, W1r, W2l, b2l, W2r, W3l, b3l, W3r, Wlin, blin):
    raise NotImplementedError("write your pallas kernel here")


# speedup vs baseline: 4.8040x; 4.8040x over previous
import functools

import jax
import jax.numpy as jnp
from jax import lax
from jax.experimental import pallas as pl
from jax.experimental.pallas import tpu as pltpu
from jax.experimental.pallas import tpu_sc as plsc

N = 10000
NPAD = 10240
E = 320000
D = 128
NC = 2
NS = 16
NW = NC * NS
EDGES_PER_W = E // NW
CHUNK = 80
NCHUNK = EDGES_PER_W // CHUNK
ROWS_PER_TILE = NPAD // NS

_MESH = plsc.VectorSubcoreMesh(core_axis_name="c", subcore_axis_name="s")


def _sc_agg_body(h_hbm, src_hbm, dst_hbm, zeros_hbm, part_hbm,
                 acc, src_v, dst_v, rows_v, sem):
    c = lax.axis_index("c")
    s = lax.axis_index("s")
    wid = s * NC + c
    r0 = s * ROWS_PER_TILE

    pltpu.sync_copy(zeros_hbm.at[pl.ds(r0, ROWS_PER_TILE)],
                    acc.at[pl.ds(r0, ROWS_PER_TILE)])
    plsc.subcore_barrier()

    base = wid * EDGES_PER_W

    def body(j, carry):
        off = base + j * CHUNK
        pltpu.sync_copy(src_hbm.at[pl.ds(off, CHUNK)], src_v)
        pltpu.sync_copy(dst_hbm.at[pl.ds(off, CHUNK)], dst_v)
        pltpu.async_copy(h_hbm.at[src_v], rows_v, sem).wait()
        pltpu.sync_copy(rows_v, acc.at[dst_v], add=True)
        return carry

    lax.fori_loop(0, NCHUNK, body, 0)

    plsc.subcore_barrier()
    pltpu.sync_copy(acc.at[pl.ds(r0, ROWS_PER_TILE)],
                    part_hbm.at[c, pl.ds(r0, ROWS_PER_TILE)])


_sc_agg = pl.kernel(
    _sc_agg_body,
    out_type=jax.ShapeDtypeStruct((NC, NPAD, D), jnp.float32),
    mesh=_MESH,
    scratch_types=[
        pltpu.VMEM_SHARED((NPAD, D), jnp.float32),
        pltpu.VMEM((CHUNK,), jnp.int32),
        pltpu.VMEM((CHUNK,), jnp.int32),
        pltpu.VMEM((CHUNK, D), jnp.float32),
        pltpu.SemaphoreType.DMA,
    ],
)


def _sc_count_body(dst_hbm, zeros_hbm, ones_hbm, cntp_hbm,
                   acc, dst_v, ones_v, sem):
    c = lax.axis_index("c")
    s = lax.axis_index("s")
    wid = s * NC + c
    r0 = s * ROWS_PER_TILE

    pltpu.sync_copy(zeros_hbm.at[pl.ds(r0, ROWS_PER_TILE)],
                    acc.at[pl.ds(r0, ROWS_PER_TILE)])
    pltpu.sync_copy(ones_hbm, ones_v)
    plsc.subcore_barrier()

    base = wid * EDGES_PER_W

    def body(j, carry):
        off = base + j * CHUNK
        pltpu.sync_copy(dst_hbm.at[pl.ds(off, CHUNK)], dst_v)
        pltpu.sync_copy(ones_v, acc.at[dst_v], add=True)
        return carry

    lax.fori_loop(0, NCHUNK, body, 0)

    plsc.subcore_barrier()
    pltpu.sync_copy(acc.at[pl.ds(r0, ROWS_PER_TILE)],
                    cntp_hbm.at[c, pl.ds(r0, ROWS_PER_TILE)])


_sc_count = pl.kernel(
    _sc_count_body,
    out_type=jax.ShapeDtypeStruct((NC, NPAD, D), jnp.float32),
    mesh=_MESH,
    scratch_types=[
        pltpu.VMEM_SHARED((NPAD, D), jnp.float32),
        pltpu.VMEM((CHUNK,), jnp.int32),
        pltpu.VMEM((CHUNK, D), jnp.float32),
        pltpu.SemaphoreType.DMA,
    ],
)


ROW_BLK = 1000
NBLK = N // ROW_BLK


def _dense_layer_body(cntp_ref, p_ref, h_ref, wl_ref, wr_ref, bl_ref, o_ref):
    cnt = cntp_ref[0, :, 0:1] + cntp_ref[1, :, 0:1]
    inv = 1.0 / jnp.maximum(cnt, 1.0)
    agg = (p_ref[0] + p_ref[1]) * inv
    out = lax.dot_general(agg, wl_ref[...], (((1,), (1,)), ((), ())),
                          preferred_element_type=jnp.float32)
    out = out + lax.dot_general(h_ref[...], wr_ref[...], (((1,), (1,)), ((), ())),
                                preferred_element_type=jnp.float32)
    o_ref[...] = jnp.maximum(out + bl_ref[...], 0.0)


def _dense_layer(cntp, p, h, wl, bl, wr):
    return pl.pallas_call(
        _dense_layer_body,
        grid=(NBLK,),
        in_specs=[
            pl.BlockSpec((NC, ROW_BLK, D), lambda i: (0, i, 0)),
            pl.BlockSpec((NC, ROW_BLK, D), lambda i: (0, i, 0)),
            pl.BlockSpec((ROW_BLK, D), lambda i: (i, 0)),
            pl.BlockSpec((D, D), lambda i: (0, 0)),
            pl.BlockSpec((D, D), lambda i: (0, 0)),
            pl.BlockSpec((1, D), lambda i: (0, 0)),
        ],
        out_specs=pl.BlockSpec((ROW_BLK, D), lambda i: (i, 0)),
        out_shape=jax.ShapeDtypeStruct((N, D), jnp.float32),
    )(cntp, p, h, wl, wr, bl.reshape(1, D))


def _head_body(G, A, h_ref, b_ref, wlin_ref, blin_ref, o_ref, sum_ref, cnt_ref):
    i = pl.program_id(0)
    t = lax.dot_general(h_ref[...], wlin_ref[...], (((1,), (1,)), ((), ())),
                        preferred_element_type=jnp.float32)
    t = jnp.tanh(t + blin_ref[...])
    ids = b_ref[0]
    onehot = (lax.broadcasted_iota(jnp.int32, (G, ROW_BLK), 0) == ids
              ).astype(jnp.float32)
    part = lax.dot_general(onehot, t, (((1,), (0,)), ((), ())),
                           preferred_element_type=jnp.float32)
    cpart = jnp.sum(onehot, axis=1, keepdims=True)

    @pl.when(i == 0)
    def _():
        sum_ref[...] = part
        cnt_ref[...] = cpart

    @pl.when(i > 0)
    def _():
        sum_ref[...] = sum_ref[...] + part
        cnt_ref[...] = cnt_ref[...] + cpart

    @pl.when(i == NBLK - 1)
    def _():
        o_ref[...] = sum_ref[...] / jnp.maximum(cnt_ref[...], 1.0)


def _head(h, batch3, wlin, blin):
    G = 64
    A = wlin.shape[0]
    return pl.pallas_call(
        functools.partial(_head_body, G, A),
        grid=(NBLK,),
        in_specs=[
            pl.BlockSpec((ROW_BLK, D), lambda i: (i, 0)),
            pl.BlockSpec((1, 1, ROW_BLK), lambda i: (i, 0, 0)),
            pl.BlockSpec((A, D), lambda i: (0, 0)),
            pl.BlockSpec((1, A), lambda i: (0, 0)),
        ],
        out_specs=pl.BlockSpec((G, A), lambda i: (0, 0)),
        out_shape=jax.ShapeDtypeStruct((G, A), jnp.float32),
        scratch_shapes=[
            pltpu.VMEM((G, A), jnp.float32),
            pltpu.VMEM((G, 1), jnp.float32),
        ],
    )(h, batch3, wlin, blin.reshape(1, A))


def kernel(x, edge_index, batch, W1l, b1l, W1r, W2l, b2l, W2r,
           W3l, b3l, W3r, Wlin, blin):
    src = edge_index[0]
    dst = edge_index[1]
    zeros = jnp.zeros((NPAD, D), jnp.float32)
    ones = jnp.ones((CHUNK, D), jnp.float32)
    batch3 = batch.reshape(NBLK, 1, ROW_BLK)

    cntp = _sc_count(dst, zeros, ones)
    p1 = _sc_agg(x, src, dst, zeros)
    h1 = _dense_layer(cntp, p1, x, W1l, b1l, W1r)
    p2 = _sc_agg(h1, src, dst, zeros)
    h2 = _dense_layer(cntp, p2, h1, W2l, b2l, W2r)
    p3 = _sc_agg(h2, src, dst, zeros)
    h3 = _dense_layer(cntp, p3, h2, W3l, b3l, W3r)
    return _head(h3, batch3, Wlin, blin)


# speedup vs baseline: 8.7157x; 1.8143x over previous
import functools

import jax
import jax.numpy as jnp
from jax import lax
from jax.experimental import pallas as pl
from jax.experimental.pallas import tpu as pltpu
from jax.experimental.pallas import tpu_sc as plsc

N = 10000
NPAD = 10240
E = 320000
D = 128
NC = 2
NS = 16
NW = NC * NS
EDGES_PER_W = E // NW
CHUNK = 80
NCHUNK = EDGES_PER_W // CHUNK
ROWS_PER_TILE = NPAD // NS

_MESH = plsc.VectorSubcoreMesh(core_axis_name="c", subcore_axis_name="s")


def _sc_agg_body(h_hbm, idx_hbm, zeros_hbm, part_hbm,
                 acc, ib0, ib1, rows0, rows1, semi0, semi1, semg0, semg1):
    c = lax.axis_index("c")
    s = lax.axis_index("s")
    wid = s * NC + c
    r0 = s * ROWS_PER_TILE

    pltpu.sync_copy(zeros_hbm.at[pl.ds(r0, ROWS_PER_TILE)],
                    acc.at[pl.ds(r0, ROWS_PER_TILE)])
    plsc.subcore_barrier()

    pltpu.async_copy(idx_hbm.at[wid, 0], ib0, semi0)
    pltpu.async_copy(idx_hbm.at[wid, 1], ib1, semi1)
    pltpu.make_async_copy(idx_hbm.at[wid, 0], ib0, semi0).wait()
    pltpu.async_copy(h_hbm.at[ib0.at[0]], rows0, semg0)

    def half(j, ib_a, ib_b, rows_a, rows_b, semi_a, semi_b, semg_a, semg_b):
        pltpu.make_async_copy(h_hbm.at[ib_a.at[0]], rows_a, semg_a).wait()
        pltpu.make_async_copy(idx_hbm.at[wid, 0], ib_b, semi_b).wait()
        pltpu.async_copy(h_hbm.at[ib_b.at[0]], rows_b, semg_b)
        pltpu.sync_copy(rows_a, acc.at[ib_a.at[1]], add=True)
        jn = jnp.minimum(j + 2, NCHUNK - 1)
        pltpu.async_copy(idx_hbm.at[wid, jn], ib_a, semi_a)

    def body(g, carry):
        j = 2 * g
        half(j, ib0, ib1, rows0, rows1, semi0, semi1, semg0, semg1)
        half(j + 1, ib1, ib0, rows1, rows0, semi1, semi0, semg1, semg0)
        return carry

    lax.fori_loop(0, (NCHUNK - 1) // 2, body, 0)
    pltpu.make_async_copy(h_hbm.at[ib0.at[0]], rows0, semg0).wait()
    pltpu.sync_copy(rows0, acc.at[ib0.at[1]], add=True)
    pltpu.make_async_copy(idx_hbm.at[wid, 0], ib1, semi1).wait()

    plsc.subcore_barrier()
    pltpu.sync_copy(acc.at[pl.ds(r0, ROWS_PER_TILE)],
                    part_hbm.at[c, pl.ds(r0, ROWS_PER_TILE)])


_sc_agg = pl.kernel(
    _sc_agg_body,
    out_type=jax.ShapeDtypeStruct((NC, NPAD, D), jnp.float32),
    mesh=_MESH,
    scratch_types=[
        pltpu.VMEM_SHARED((NPAD, D), jnp.float32),
        pltpu.VMEM((2, CHUNK), jnp.int32),
        pltpu.VMEM((2, CHUNK), jnp.int32),
        pltpu.VMEM((CHUNK, D), jnp.float32),
        pltpu.VMEM((CHUNK, D), jnp.float32),
        pltpu.SemaphoreType.DMA,
        pltpu.SemaphoreType.DMA,
        pltpu.SemaphoreType.DMA,
        pltpu.SemaphoreType.DMA,
    ],
)


def _sc_count_body(idx_hbm, zeros_hbm, ones_hbm, cntp_hbm,
                   acc, ib0, ib1, ones_v, semi0, semi1):
    c = lax.axis_index("c")
    s = lax.axis_index("s")
    wid = s * NC + c
    r0 = s * ROWS_PER_TILE

    pltpu.sync_copy(zeros_hbm.at[pl.ds(r0, ROWS_PER_TILE)],
                    acc.at[pl.ds(r0, ROWS_PER_TILE)])
    pltpu.sync_copy(ones_hbm, ones_v)
    plsc.subcore_barrier()

    pltpu.async_copy(idx_hbm.at[wid, 0], ib0, semi0)
    pltpu.async_copy(idx_hbm.at[wid, 1], ib1, semi1)

    def half(j, ib_a, ib_b, semi_a, semi_b):
        pltpu.make_async_copy(idx_hbm.at[wid, 0], ib_a, semi_a).wait()
        pltpu.sync_copy(ones_v, acc.at[ib_a.at[1]], add=True)
        jn = jnp.minimum(j + 2, NCHUNK - 1)
        pltpu.async_copy(idx_hbm.at[wid, jn], ib_a, semi_a)

    def body(g, carry):
        j = 2 * g
        half(j, ib0, ib1, semi0, semi1)
        half(j + 1, ib1, ib0, semi1, semi0)
        return carry

    lax.fori_loop(0, (NCHUNK - 1) // 2, body, 0)
    pltpu.make_async_copy(idx_hbm.at[wid, 0], ib0, semi0).wait()
    pltpu.sync_copy(ones_v, acc.at[ib0.at[1]], add=True)
    pltpu.make_async_copy(idx_hbm.at[wid, 0], ib1, semi1).wait()

    plsc.subcore_barrier()
    pltpu.sync_copy(acc.at[pl.ds(r0, ROWS_PER_TILE)],
                    cntp_hbm.at[c, pl.ds(r0, ROWS_PER_TILE)])


_sc_count = pl.kernel(
    _sc_count_body,
    out_type=jax.ShapeDtypeStruct((NC, NPAD, D), jnp.float32),
    mesh=_MESH,
    scratch_types=[
        pltpu.VMEM_SHARED((NPAD, D), jnp.float32),
        pltpu.VMEM((2, CHUNK), jnp.int32),
        pltpu.VMEM((2, CHUNK), jnp.int32),
        pltpu.VMEM((CHUNK, D), jnp.float32),
        pltpu.SemaphoreType.DMA,
        pltpu.SemaphoreType.DMA,
    ],
)


ROW_BLK = 1000
NBLK = N // ROW_BLK


def _dense_layer_body(cntp_ref, p_ref, h_ref, wl_ref, wr_ref, bl_ref, o_ref):
    cnt = cntp_ref[0, :, 0:1] + cntp_ref[1, :, 0:1]
    inv = 1.0 / jnp.maximum(cnt, 1.0)
    agg = (p_ref[0] + p_ref[1]) * inv
    out = lax.dot_general(agg, wl_ref[...], (((1,), (1,)), ((), ())),
                          preferred_element_type=jnp.float32)
    out = out + lax.dot_general(h_ref[...], wr_ref[...], (((1,), (1,)), ((), ())),
                                preferred_element_type=jnp.float32)
    o_ref[...] = jnp.maximum(out + bl_ref[...], 0.0)


def _dense_layer(cntp, p, h, wl, bl, wr):
    return pl.pallas_call(
        _dense_layer_body,
        grid=(NBLK,),
        in_specs=[
            pl.BlockSpec((NC, ROW_BLK, D), lambda i: (0, i, 0)),
            pl.BlockSpec((NC, ROW_BLK, D), lambda i: (0, i, 0)),
            pl.BlockSpec((ROW_BLK, D), lambda i: (i, 0)),
            pl.BlockSpec((D, D), lambda i: (0, 0)),
            pl.BlockSpec((D, D), lambda i: (0, 0)),
            pl.BlockSpec((1, D), lambda i: (0, 0)),
        ],
        out_specs=pl.BlockSpec((ROW_BLK, D), lambda i: (i, 0)),
        out_shape=jax.ShapeDtypeStruct((N, D), jnp.float32),
    )(cntp, p, h, wl, wr, bl.reshape(1, D))


def _head_body(G, A, h_ref, b_ref, wlin_ref, blin_ref, o_ref, sum_ref, cnt_ref):
    i = pl.program_id(0)
    t = lax.dot_general(h_ref[...], wlin_ref[...], (((1,), (1,)), ((), ())),
                        preferred_element_type=jnp.float32)
    t = jnp.tanh(t + blin_ref[...])
    ids = b_ref[0]
    onehot = (lax.broadcasted_iota(jnp.int32, (G, ROW_BLK), 0) == ids
              ).astype(jnp.float32)
    part = lax.dot_general(onehot, t, (((1,), (0,)), ((), ())),
                           preferred_element_type=jnp.float32)
    cpart = jnp.sum(onehot, axis=1, keepdims=True)

    @pl.when(i == 0)
    def _():
        sum_ref[...] = part
        cnt_ref[...] = cpart

    @pl.when(i > 0)
    def _():
        sum_ref[...] = sum_ref[...] + part
        cnt_ref[...] = cnt_ref[...] + cpart

    @pl.when(i == NBLK - 1)
    def _():
        o_ref[...] = sum_ref[...] / jnp.maximum(cnt_ref[...], 1.0)


def _head(h, batch3, wlin, blin):
    G = 64
    A = wlin.shape[0]
    return pl.pallas_call(
        functools.partial(_head_body, G, A),
        grid=(NBLK,),
        in_specs=[
            pl.BlockSpec((ROW_BLK, D), lambda i: (i, 0)),
            pl.BlockSpec((1, 1, ROW_BLK), lambda i: (i, 0, 0)),
            pl.BlockSpec((A, D), lambda i: (0, 0)),
            pl.BlockSpec((1, A), lambda i: (0, 0)),
        ],
        out_specs=pl.BlockSpec((G, A), lambda i: (0, 0)),
        out_shape=jax.ShapeDtypeStruct((G, A), jnp.float32),
        scratch_shapes=[
            pltpu.VMEM((G, A), jnp.float32),
            pltpu.VMEM((G, 1), jnp.float32),
        ],
    )(h, batch3, wlin, blin.reshape(1, A))


def kernel(x, edge_index, batch, W1l, b1l, W1r, W2l, b2l, W2r,
           W3l, b3l, W3r, Wlin, blin):
    idx = jnp.transpose(edge_index.reshape(2, NW, NCHUNK, CHUNK), (1, 2, 0, 3))
    zeros = jnp.zeros((NPAD, D), jnp.float32)
    ones = jnp.ones((CHUNK, D), jnp.float32)
    batch3 = batch.reshape(NBLK, 1, ROW_BLK)

    cntp = _sc_count(idx, zeros, ones)
    p1 = _sc_agg(x, idx, zeros)
    h1 = _dense_layer(cntp, p1, x, W1l, b1l, W1r)
    p2 = _sc_agg(h1, idx, zeros)
    h2 = _dense_layer(cntp, p2, h1, W2l, b2l, W2r)
    p3 = _sc_agg(h2, idx, zeros)
    h3 = _dense_layer(cntp, p3, h2, W3l, b3l, W3r)
    return _head(h3, batch3, Wlin, blin)


# speedup vs baseline: 8.7180x; 1.0003x over previous
import functools

import jax
import jax.numpy as jnp
from jax import lax
from jax.experimental import pallas as pl
from jax.experimental.pallas import tpu as pltpu
from jax.experimental.pallas import tpu_sc as plsc

N = 10000
NPAD = 10240
E = 320000
D = 128
NC = 2
NS = 16
NW = NC * NS
EDGES_PER_W = E // NW
CHUNK = 80
NCHUNK = EDGES_PER_W // CHUNK
ROWS_PER_TILE = NPAD // NS

_MESH = plsc.VectorSubcoreMesh(core_axis_name="c", subcore_axis_name="s")


def _sc_agg_body(h_hbm, idx_hbm, zeros_hbm, part_hbm,
                 acc, ib0, ib1, ib2, ib3, rows0, rows1,
                 semi0, semi1, semi2, semi3, semg0, semg1, sems0, sems1):
    c = lax.axis_index("c")
    s = lax.axis_index("s")
    wid = s * NC + c
    r0 = s * ROWS_PER_TILE

    pltpu.sync_copy(zeros_hbm.at[pl.ds(r0, ROWS_PER_TILE)],
                    acc.at[pl.ds(r0, ROWS_PER_TILE)])
    plsc.subcore_barrier()

    ibs = (ib0, ib1, ib2, ib3)
    rows = (rows0, rows1)
    semi = (semi0, semi1, semi2, semi3)
    semg = (semg0, semg1)
    sems = (sems0, sems1)

    def idx_load(j, jb):
        pltpu.async_copy(idx_hbm.at[wid, j], ibs[jb], semi[jb])

    def idx_wait(jb):
        pltpu.make_async_copy(idx_hbm.at[wid, 0], ibs[jb], semi[jb]).wait()

    def gather(j_ib, p):
        pltpu.async_copy(h_hbm.at[ibs[j_ib].at[0]], rows[p], semg[p])

    def gather_wait(p):
        pltpu.make_async_copy(h_hbm.at[ib0.at[0]], rows[p], semg[p]).wait()

    def scatter(jb, p):
        pltpu.async_copy(rows[p], acc.at[ibs[jb].at[1]], sems[p], add=True)

    def scatter_wait(p):
        pltpu.make_async_copy(h_hbm.at[ib0.at[0]], rows[p], sems[p]).wait()

    def chunk_body(j, p, jb, first, do_next_gather, do_idx_load):
        gather_wait(p)
        scatter(jb, p)
        if not first:
            scatter_wait(1 - p)
        if do_next_gather:
            idx_wait((jb + 1) % 4)
            gather((jb + 1) % 4, 1 - p)
        if do_idx_load:
            idx_load(j + 3, (jb + 3) % 4)

    idx_load(0, 0)
    idx_load(1, 1)
    idx_load(2, 2)
    idx_wait(0)
    gather(0, 0)
    chunk_body(0, 0, 0, True, True, True)

    def body(g, carry):
        j = 4 * g + 1
        chunk_body(j, 1, 1, False, True, True)
        chunk_body(j + 1, 0, 2, False, True, True)
        chunk_body(j + 2, 1, 3, False, True, True)
        chunk_body(j + 3, 0, 0, False, True, True)
        return carry

    lax.fori_loop(0, (NCHUNK - 5) // 4, body, 0)
    chunk_body(121, 1, 1, False, True, True)
    chunk_body(122, 0, 2, False, True, False)
    chunk_body(123, 1, 3, False, True, False)
    chunk_body(124, 0, 0, False, False, False)
    scatter_wait(0)

    plsc.subcore_barrier()
    pltpu.sync_copy(acc.at[pl.ds(r0, ROWS_PER_TILE)],
                    part_hbm.at[c, pl.ds(r0, ROWS_PER_TILE)])


_sc_agg = pl.kernel(
    _sc_agg_body,
    out_type=jax.ShapeDtypeStruct((NC, NPAD, D), jnp.float32),
    mesh=_MESH,
    scratch_types=[
        pltpu.VMEM_SHARED((NPAD, D), jnp.float32),
        pltpu.VMEM((2, CHUNK), jnp.int32),
        pltpu.VMEM((2, CHUNK), jnp.int32),
        pltpu.VMEM((2, CHUNK), jnp.int32),
        pltpu.VMEM((2, CHUNK), jnp.int32),
        pltpu.VMEM((CHUNK, D), jnp.float32),
        pltpu.VMEM((CHUNK, D), jnp.float32),
        pltpu.SemaphoreType.DMA,
        pltpu.SemaphoreType.DMA,
        pltpu.SemaphoreType.DMA,
        pltpu.SemaphoreType.DMA,
        pltpu.SemaphoreType.DMA,
        pltpu.SemaphoreType.DMA,
        pltpu.SemaphoreType.DMA,
        pltpu.SemaphoreType.DMA,
    ],
)


def _sc_count_body(idx_hbm, zeros_hbm, ones_hbm, cntp_hbm,
                   acc, ib0, ib1, ones_v, semi0, semi1):
    c = lax.axis_index("c")
    s = lax.axis_index("s")
    wid = s * NC + c
    r0 = s * ROWS_PER_TILE

    pltpu.sync_copy(zeros_hbm.at[pl.ds(r0, ROWS_PER_TILE)],
                    acc.at[pl.ds(r0, ROWS_PER_TILE)])
    pltpu.sync_copy(ones_hbm, ones_v)
    plsc.subcore_barrier()

    pltpu.async_copy(idx_hbm.at[wid, 0], ib0, semi0)
    pltpu.async_copy(idx_hbm.at[wid, 1], ib1, semi1)

    def half(j, ib_a, ib_b, semi_a, semi_b):
        pltpu.make_async_copy(idx_hbm.at[wid, 0], ib_a, semi_a).wait()
        pltpu.sync_copy(ones_v, acc.at[ib_a.at[1]], add=True)
        jn = jnp.minimum(j + 2, NCHUNK - 1)
        pltpu.async_copy(idx_hbm.at[wid, jn], ib_a, semi_a)

    def body(g, carry):
        j = 2 * g
        half(j, ib0, ib1, semi0, semi1)
        half(j + 1, ib1, ib0, semi1, semi0)
        return carry

    lax.fori_loop(0, (NCHUNK - 1) // 2, body, 0)
    pltpu.make_async_copy(idx_hbm.at[wid, 0], ib0, semi0).wait()
    pltpu.sync_copy(ones_v, acc.at[ib0.at[1]], add=True)
    pltpu.make_async_copy(idx_hbm.at[wid, 0], ib1, semi1).wait()

    plsc.subcore_barrier()
    pltpu.sync_copy(acc.at[pl.ds(r0, ROWS_PER_TILE)],
                    cntp_hbm.at[c, pl.ds(r0, ROWS_PER_TILE)])


_sc_count = pl.kernel(
    _sc_count_body,
    out_type=jax.ShapeDtypeStruct((NC, NPAD, D), jnp.float32),
    mesh=_MESH,
    scratch_types=[
        pltpu.VMEM_SHARED((NPAD, D), jnp.float32),
        pltpu.VMEM((2, CHUNK), jnp.int32),
        pltpu.VMEM((2, CHUNK), jnp.int32),
        pltpu.VMEM((CHUNK, D), jnp.float32),
        pltpu.SemaphoreType.DMA,
        pltpu.SemaphoreType.DMA,
    ],
)


ROW_BLK = 1000
NBLK = N // ROW_BLK


def _dense_layer_body(cntp_ref, p_ref, h_ref, wl_ref, wr_ref, bl_ref, o_ref):
    cnt = cntp_ref[0, :, 0:1] + cntp_ref[1, :, 0:1]
    inv = 1.0 / jnp.maximum(cnt, 1.0)
    agg = (p_ref[0] + p_ref[1]) * inv
    out = lax.dot_general(agg, wl_ref[...], (((1,), (1,)), ((), ())),
                          preferred_element_type=jnp.float32)
    out = out + lax.dot_general(h_ref[...], wr_ref[...], (((1,), (1,)), ((), ())),
                                preferred_element_type=jnp.float32)
    o_ref[...] = jnp.maximum(out + bl_ref[...], 0.0)


def _dense_layer(cntp, p, h, wl, bl, wr):
    return pl.pallas_call(
        _dense_layer_body,
        grid=(NBLK,),
        in_specs=[
            pl.BlockSpec((NC, ROW_BLK, D), lambda i: (0, i, 0)),
            pl.BlockSpec((NC, ROW_BLK, D), lambda i: (0, i, 0)),
            pl.BlockSpec((ROW_BLK, D), lambda i: (i, 0)),
            pl.BlockSpec((D, D), lambda i: (0, 0)),
            pl.BlockSpec((D, D), lambda i: (0, 0)),
            pl.BlockSpec((1, D), lambda i: (0, 0)),
        ],
        out_specs=pl.BlockSpec((ROW_BLK, D), lambda i: (i, 0)),
        out_shape=jax.ShapeDtypeStruct((N, D), jnp.float32),
    )(cntp, p, h, wl, wr, bl.reshape(1, D))


def _head_body(G, A, h_ref, b_ref, wlin_ref, blin_ref, o_ref, sum_ref, cnt_ref):
    i = pl.program_id(0)
    t = lax.dot_general(h_ref[...], wlin_ref[...], (((1,), (1,)), ((), ())),
                        preferred_element_type=jnp.float32)
    t = jnp.tanh(t + blin_ref[...])
    ids = b_ref[0]
    onehot = (lax.broadcasted_iota(jnp.int32, (G, ROW_BLK), 0) == ids
              ).astype(jnp.float32)
    part = lax.dot_general(onehot, t, (((1,), (0,)), ((), ())),
                           preferred_element_type=jnp.float32)
    cpart = jnp.sum(onehot, axis=1, keepdims=True)

    @pl.when(i == 0)
    def _():
        sum_ref[...] = part
        cnt_ref[...] = cpart

    @pl.when(i > 0)
    def _():
        sum_ref[...] = sum_ref[...] + part
        cnt_ref[...] = cnt_ref[...] + cpart

    @pl.when(i == NBLK - 1)
    def _():
        o_ref[...] = sum_ref[...] / jnp.maximum(cnt_ref[...], 1.0)


def _head(h, batch3, wlin, blin):
    G = 64
    A = wlin.shape[0]
    return pl.pallas_call(
        functools.partial(_head_body, G, A),
        grid=(NBLK,),
        in_specs=[
            pl.BlockSpec((ROW_BLK, D), lambda i: (i, 0)),
            pl.BlockSpec((1, 1, ROW_BLK), lambda i: (i, 0, 0)),
            pl.BlockSpec((A, D), lambda i: (0, 0)),
            pl.BlockSpec((1, A), lambda i: (0, 0)),
        ],
        out_specs=pl.BlockSpec((G, A), lambda i: (0, 0)),
        out_shape=jax.ShapeDtypeStruct((G, A), jnp.float32),
        scratch_shapes=[
            pltpu.VMEM((G, A), jnp.float32),
            pltpu.VMEM((G, 1), jnp.float32),
        ],
    )(h, batch3, wlin, blin.reshape(1, A))


def kernel(x, edge_index, batch, W1l, b1l, W1r, W2l, b2l, W2r,
           W3l, b3l, W3r, Wlin, blin):
    idx = jnp.transpose(edge_index.reshape(2, NW, NCHUNK, CHUNK), (1, 2, 0, 3))
    zeros = jnp.zeros((NPAD, D), jnp.float32)
    ones = jnp.ones((CHUNK, D), jnp.float32)
    batch3 = batch.reshape(NBLK, 1, ROW_BLK)

    cntp = _sc_count(idx, zeros, ones)
    p1 = _sc_agg(x, idx, zeros)
    h1 = _dense_layer(cntp, p1, x, W1l, b1l, W1r)
    p2 = _sc_agg(h1, idx, zeros)
    h2 = _dense_layer(cntp, p2, h1, W2l, b2l, W2r)
    p3 = _sc_agg(h2, idx, zeros)
    h3 = _dense_layer(cntp, p3, h2, W3l, b3l, W3r)
    return _head(h3, batch3, Wlin, blin)


# speedup vs baseline: 9.6609x; 1.1082x over previous
import functools

import jax
import jax.numpy as jnp
from jax import lax
from jax.experimental import pallas as pl
from jax.experimental.pallas import tpu as pltpu
from jax.experimental.pallas import tpu_sc as plsc

N = 10000
NPAD = 10240
E = 320000
D = 128
NC = 2
NS = 16
NW = NC * NS
EDGES_PER_W = E // NW
CHUNK = 80
NCHUNK = EDGES_PER_W // CHUNK
ROWS_PER_TILE = NPAD // NS

_MESH = plsc.VectorSubcoreMesh(core_axis_name="c", subcore_axis_name="s")


def _sc_agg_body(h_hbm, idx_hbm, zeros_hbm, part_hbm,
                 acc, ib0, ib1, ib2, ib3, rows0, rows1,
                 semi0, semi1, semi2, semi3, semg0, semg1, sems0, sems1):
    c = lax.axis_index("c")
    s = lax.axis_index("s")
    wid = s * NC + c
    r0 = s * ROWS_PER_TILE

    pltpu.sync_copy(zeros_hbm.at[pl.ds(r0, ROWS_PER_TILE)],
                    acc.at[pl.ds(r0, ROWS_PER_TILE)])
    plsc.subcore_barrier()

    ibs = (ib0, ib1, ib2, ib3)
    rows = (rows0, rows1)
    semi = (semi0, semi1, semi2, semi3)
    semg = (semg0, semg1)
    sems = (sems0, sems1)

    def idx_load(j, jb):
        pltpu.async_copy(idx_hbm.at[wid, j], ibs[jb], semi[jb])

    def idx_wait(jb):
        pltpu.make_async_copy(idx_hbm.at[wid, 0], ibs[jb], semi[jb]).wait()

    def gather(j_ib, p):
        pltpu.async_copy(h_hbm.at[ibs[j_ib].at[0]], rows[p], semg[p])

    def gather_wait(p):
        pltpu.make_async_copy(h_hbm.at[ib0.at[0]], rows[p], semg[p]).wait()

    def scatter(jb, p):
        pltpu.async_copy(rows[p], acc.at[ibs[jb].at[1]], sems[p], add=True)

    def scatter_wait(p):
        pltpu.make_async_copy(h_hbm.at[ib0.at[0]], rows[p], sems[p]).wait()

    def chunk_body(j, p, jb, first, do_next_gather, do_idx_load):
        gather_wait(p)
        scatter(jb, p)
        if not first:
            scatter_wait(1 - p)
        if do_next_gather:
            idx_wait((jb + 1) % 4)
            gather((jb + 1) % 4, 1 - p)
        if do_idx_load:
            idx_load(j + 3, (jb + 3) % 4)

    idx_load(0, 0)
    idx_load(1, 1)
    idx_load(2, 2)
    idx_wait(0)
    gather(0, 0)
    chunk_body(0, 0, 0, True, True, True)

    def body(g, carry):
        j = 4 * g + 1
        chunk_body(j, 1, 1, False, True, True)
        chunk_body(j + 1, 0, 2, False, True, True)
        chunk_body(j + 2, 1, 3, False, True, True)
        chunk_body(j + 3, 0, 0, False, True, True)
        return carry

    lax.fori_loop(0, (NCHUNK - 5) // 4, body, 0)
    chunk_body(121, 1, 1, False, True, True)
    chunk_body(122, 0, 2, False, True, False)
    chunk_body(123, 1, 3, False, True, False)
    chunk_body(124, 0, 0, False, False, False)
    scatter_wait(0)

    plsc.subcore_barrier()
    pltpu.sync_copy(acc.at[pl.ds(r0, ROWS_PER_TILE)],
                    part_hbm.at[c, pl.ds(r0, ROWS_PER_TILE)])


_sc_agg = pl.kernel(
    _sc_agg_body,
    out_type=jax.ShapeDtypeStruct((NC, NPAD, D), jnp.float32),
    mesh=_MESH,
    scratch_types=[
        pltpu.VMEM_SHARED((NPAD, D), jnp.float32),
        pltpu.VMEM((2, CHUNK), jnp.int32),
        pltpu.VMEM((2, CHUNK), jnp.int32),
        pltpu.VMEM((2, CHUNK), jnp.int32),
        pltpu.VMEM((2, CHUNK), jnp.int32),
        pltpu.VMEM((CHUNK, D), jnp.float32),
        pltpu.VMEM((CHUNK, D), jnp.float32),
        pltpu.SemaphoreType.DMA,
        pltpu.SemaphoreType.DMA,
        pltpu.SemaphoreType.DMA,
        pltpu.SemaphoreType.DMA,
        pltpu.SemaphoreType.DMA,
        pltpu.SemaphoreType.DMA,
        pltpu.SemaphoreType.DMA,
        pltpu.SemaphoreType.DMA,
    ],
)


EDGES_PER_TILE = E // NS
NVREG = EDGES_PER_TILE // 16
STRIPE = NPAD // NS
HALF_STRIPE = STRIPE // NC


def _sc_inv_body(dst_hbm, zeros1_hbm, inv_hbm, shared, hist, dbuf, mbuf, bbuf, sem):
    c = lax.axis_index("c")
    s = lax.axis_index("s")
    pltpu.sync_copy(dst_hbm.at[pl.ds(s * EDGES_PER_TILE, EDGES_PER_TILE)], dbuf)
    pltpu.sync_copy(zeros1_hbm, hist)
    ones16 = jnp.ones((16,), jnp.float32)

    def hbody(v, carry):
        iv = dbuf[pl.ds(v * 16, 16)]
        plsc.addupdate_scatter(hist, [iv], ones16)
        return carry

    lax.fori_loop(0, NVREG, hbody, 0)
    pltpu.sync_copy(hist, shared.at[s])
    plsc.subcore_barrier()

    pltpu.sync_copy(shared.at[:, pl.ds(s * STRIPE, STRIPE)], mbuf)

    def mbody(kk, carry):
        base = c * HALF_STRIPE + kk * 16
        acc16 = mbuf[0, pl.ds(base, 16)]
        for r in range(1, NS):
            acc16 = acc16 + mbuf[r, pl.ds(base, 16)]
        inv16 = 1.0 / jnp.maximum(acc16, 1.0)
        for l in range(16):
            row = jnp.full((16,), inv16[l], jnp.float32)
            for cb in range(D // 16):
                bbuf[kk * 16 + l, pl.ds(cb * 16, 16)] = row
        return carry

    lax.fori_loop(0, HALF_STRIPE // 16, mbody, 0)
    pltpu.sync_copy(bbuf, inv_hbm.at[pl.ds(s * STRIPE + c * HALF_STRIPE,
                                           HALF_STRIPE)])


_sc_inv = pl.kernel(
    _sc_inv_body,
    out_type=jax.ShapeDtypeStruct((NPAD, D), jnp.float32),
    mesh=_MESH,
    compiler_params=pltpu.CompilerParams(needs_layout_passes=False),
    scratch_types=[
        pltpu.VMEM_SHARED((NS, NPAD), jnp.float32),
        pltpu.VMEM((NPAD,), jnp.float32),
        pltpu.VMEM((EDGES_PER_TILE,), jnp.int32),
        pltpu.VMEM((NS, STRIPE), jnp.float32),
        pltpu.VMEM((HALF_STRIPE, D), jnp.float32),
        pltpu.SemaphoreType.DMA,
    ],
)


ROW_BLK = 1000
NBLK = N // ROW_BLK


def _dense_layer_body(inv_ref, p_ref, h_ref, wl_ref, wr_ref, bl_ref, o_ref):
    agg = (p_ref[0] + p_ref[1]) * inv_ref[...]
    out = lax.dot_general(agg, wl_ref[...], (((1,), (1,)), ((), ())),
                          preferred_element_type=jnp.float32)
    out = out + lax.dot_general(h_ref[...], wr_ref[...], (((1,), (1,)), ((), ())),
                                preferred_element_type=jnp.float32)
    o_ref[...] = jnp.maximum(out + bl_ref[...], 0.0)


def _dense_layer(inv, p, h, wl, bl, wr):
    return pl.pallas_call(
        _dense_layer_body,
        grid=(NBLK,),
        in_specs=[
            pl.BlockSpec((ROW_BLK, D), lambda i: (i, 0)),
            pl.BlockSpec((NC, ROW_BLK, D), lambda i: (0, i, 0)),
            pl.BlockSpec((ROW_BLK, D), lambda i: (i, 0)),
            pl.BlockSpec((D, D), lambda i: (0, 0)),
            pl.BlockSpec((D, D), lambda i: (0, 0)),
            pl.BlockSpec((1, D), lambda i: (0, 0)),
        ],
        out_specs=pl.BlockSpec((ROW_BLK, D), lambda i: (i, 0)),
        out_shape=jax.ShapeDtypeStruct((N, D), jnp.float32),
    )(inv, p, h, wl, wr, bl.reshape(1, D))


def _head_body(G, A, h_ref, b_ref, wlin_ref, blin_ref, o_ref, sum_ref, cnt_ref):
    i = pl.program_id(0)
    t = lax.dot_general(h_ref[...], wlin_ref[...], (((1,), (1,)), ((), ())),
                        preferred_element_type=jnp.float32)
    t = jnp.tanh(t + blin_ref[...])
    ids = b_ref[0]
    onehot = (lax.broadcasted_iota(jnp.int32, (G, ROW_BLK), 0) == ids
              ).astype(jnp.float32)
    part = lax.dot_general(onehot, t, (((1,), (0,)), ((), ())),
                           preferred_element_type=jnp.float32)
    cpart = jnp.sum(onehot, axis=1, keepdims=True)

    @pl.when(i == 0)
    def _():
        sum_ref[...] = part
        cnt_ref[...] = cpart

    @pl.when(i > 0)
    def _():
        sum_ref[...] = sum_ref[...] + part
        cnt_ref[...] = cnt_ref[...] + cpart

    @pl.when(i == NBLK - 1)
    def _():
        o_ref[...] = sum_ref[...] / jnp.maximum(cnt_ref[...], 1.0)


def _head(h, batch3, wlin, blin):
    G = 64
    A = wlin.shape[0]
    return pl.pallas_call(
        functools.partial(_head_body, G, A),
        grid=(NBLK,),
        in_specs=[
            pl.BlockSpec((ROW_BLK, D), lambda i: (i, 0)),
            pl.BlockSpec((1, 1, ROW_BLK), lambda i: (i, 0, 0)),
            pl.BlockSpec((A, D), lambda i: (0, 0)),
            pl.BlockSpec((1, A), lambda i: (0, 0)),
        ],
        out_specs=pl.BlockSpec((G, A), lambda i: (0, 0)),
        out_shape=jax.ShapeDtypeStruct((G, A), jnp.float32),
        scratch_shapes=[
            pltpu.VMEM((G, A), jnp.float32),
            pltpu.VMEM((G, 1), jnp.float32),
        ],
    )(h, batch3, wlin, blin.reshape(1, A))


def kernel(x, edge_index, batch, W1l, b1l, W1r, W2l, b2l, W2r,
           W3l, b3l, W3r, Wlin, blin):
    idx = jnp.transpose(edge_index.reshape(2, NW, NCHUNK, CHUNK), (1, 2, 0, 3))
    zeros = jnp.zeros((NPAD, D), jnp.float32)
    zeros1 = jnp.zeros((NPAD,), jnp.float32)
    batch3 = batch.reshape(NBLK, 1, ROW_BLK)

    inv = _sc_inv(edge_index[1], zeros1)
    p1 = _sc_agg(x, idx, zeros)
    h1 = _dense_layer(inv, p1, x, W1l, b1l, W1r)
    p2 = _sc_agg(h1, idx, zeros)
    h2 = _dense_layer(inv, p2, h1, W2l, b2l, W2r)
    p3 = _sc_agg(h2, idx, zeros)
    h3 = _dense_layer(inv, p3, h2, W3l, b3l, W3r)
    return _head(h3, batch3, Wlin, blin)
